# baseline (speedup 1.0000x reference)
import functools
import jax
import jax.numpy as jnp
from jax import lax
from jax.experimental import pallas as pl
from jax.experimental.pallas import tpu as pltpu
from jax.experimental.pallas import tpu_sc as plsc

N, E, D, DE, H = 10000, 320000, 128, 16, 128
NC, NS, L = 2, 16, 16
NW = NC * NS
CH = 64
E_PAD = 323584
NCHUNKS = E_PAD // CH
CPW = NCHUNKS // NW
N_PR = N + 8
ROWS_PER_SUB = 624


def _node_prep_body(nf_ref, nh_ref, wn_ref, pr_ref):
    np_blk = lax.dot_general(nf_ref[...], wn_ref[...],
                             (((1,), (1,)), ((), ())),
                             preferred_element_type=jnp.float32)
    p = jnp.exp(np_blk - jnp.max(np_blk, axis=1, keepdims=True))
    pr_ref[:, :H] = p
    pr_ref[:, H:] = nh_ref[...] * p


def _edge_prep_body(ef_ref, we_ref, q_ref):
    ep = lax.dot_general(ef_ref[...], we_ref[...],
                         (((1,), (1,)), ((), ())),
                         preferred_element_type=jnp.float32)
    q_ref[...] = jnp.exp(ep - jnp.max(ep, axis=1, keepdims=True))


def _gru_body(hp_ref, nh_ref, wih_ref, whh_ref, bih_ref, bhh_ref, out_ref):
    h_new = hp_ref[0] + hp_ref[1]
    h = nh_ref[...]
    gi = lax.dot_general(h_new, wih_ref[...], (((1,), (1,)), ((), ())),
                         preferred_element_type=jnp.float32) + bih_ref[...]
    gh = lax.dot_general(h, whh_ref[...], (((1,), (1,)), ((), ())),
                         preferred_element_type=jnp.float32) + bhh_ref[...]
    r = jax.nn.sigmoid(gi[:, :H] + gh[:, :H])
    z = jax.nn.sigmoid(gi[:, H:2 * H] + gh[:, H:2 * H])
    n = jnp.tanh(gi[:, 2 * H:] + r * gh[:, 2 * H:])
    out_ref[...] = (1.0 - z) * n + z * h


def _sc_body(pr_hbm, q_hbm, ei_hbm, zero_hbm, out_hbm,
             src_a, dst_a, src_b, dst_b, pr_a, pr_b, q_a, q_b,
             acc_sh, sem_a, sem_b, sem_sa, sem_da, sem_sb, sem_db):
    cid = lax.axis_index("c")
    sid = lax.axis_index("s")
    wid = sid * NC + cid

    @pl.when(sid == 0)
    def _():
        pltpu.sync_copy(zero_hbm, acc_sh)

    plsc.subcore_barrier()

    def chunk_base(k):
        return (wid + k * NW) * CH

    def idx_src(k, ref, sem):
        pltpu.async_copy(ei_hbm.at[pl.ds(chunk_base(k), CH)], ref, sem)

    def idx_dst(k, ref, sem):
        pltpu.async_copy(ei_hbm.at[pl.ds(E_PAD + chunk_base(k), CH)], ref, sem)

    def wait_idx(ref, sem):
        pltpu.make_async_copy(ei_hbm.at[pl.ds(0, CH)], ref, sem).wait()

    def issue_gathers(k, src_v, pr_v, q_v, sem):
        pltpu.async_copy(pr_hbm.at[src_v], pr_v, sem)
        pltpu.async_copy(q_hbm.at[pl.ds(chunk_base(k), CH)], q_v, sem)

    def wait_gathers(src_v, pr_v, q_v, sem):
        pltpu.make_async_copy(pr_hbm.at[src_v], pr_v, sem).wait()
        pltpu.make_async_copy(q_hbm.at[pl.ds(0, CH)], q_v, sem).wait()

    lanes = lax.iota(jnp.int32, L)
    dnums = lax.GatherDimensionNumbers(
        offset_dims=(), collapsed_slice_dims=(0,), start_index_map=(0,))

    def compute_scatter(dst_v, pr_v, q_v):
        @plsc.parallel_loop(0, CH, unroll=2)
        def edge_body(i):
            qs = [q_v[i, pl.ds(j * L, L)] for j in range(H // L)]
            rqs = [pr_v[i, pl.ds(H + j * L, L)] * qs[j] for j in range(H // L)]
            acc = pr_v[i, pl.ds(0, L)] * qs[0]
            for j in range(1, H // L):
                acc = acc + pr_v[i, pl.ds(j * L, L)] * qs[j]
            for step in (1, 2, 4, 8):
                perm = lax.gather(
                    acc, (lanes ^ step).reshape(L, 1), dnums,
                    slice_sizes=(1,),
                    mode=lax.GatherScatterMode.PROMISE_IN_BOUNDS)
                acc = acc + perm
            s = 1.0 / acc
            for j in range(H // L):
                q_v[i, pl.ds(j * L, L)] = rqs[j] * s

        pltpu.sync_copy(q_v, acc_sh.at[dst_v], add=True)

    idx_src(0, src_a, sem_sa)
    idx_dst(0, dst_a, sem_da)
    wait_idx(src_a, sem_sa)
    issue_gathers(0, src_a, pr_a, q_a, sem_a)
    idx_src(1, src_b, sem_sb)
    idx_dst(1, dst_b, sem_db)

    def pair_body(kk, carry):
        k0 = 2 * kk
        wait_idx(src_b, sem_sb)
        issue_gathers(k0 + 1, src_b, pr_b, q_b, sem_b)
        wait_gathers(src_a, pr_a, q_a, sem_a)

        @pl.when(k0 + 2 < CPW)
        def _():
            idx_src(k0 + 2, src_a, sem_sa)

        wait_idx(dst_a, sem_da)
        compute_scatter(dst_a, pr_a, q_a)

        @pl.when(k0 + 2 < CPW)
        def _():
            idx_dst(k0 + 2, dst_a, sem_da)
            wait_idx(src_a, sem_sa)
            issue_gathers(k0 + 2, src_a, pr_a, q_a, sem_a)

        wait_gathers(src_b, pr_b, q_b, sem_b)

        @pl.when(k0 + 3 < CPW)
        def _():
            idx_src(k0 + 3, src_b, sem_sb)

        wait_idx(dst_b, sem_db)
        compute_scatter(dst_b, pr_b, q_b)

        @pl.when(k0 + 3 < CPW)
        def _():
            idx_dst(k0 + 3, dst_b, sem_db)

        return carry

    lax.fori_loop(0, CPW // 2, pair_body, 0)

    plsc.subcore_barrier()
    row0 = sid * ROWS_PER_SUB
    pltpu.sync_copy(acc_sh.at[pl.ds(row0, ROWS_PER_SUB)],
                    out_hbm.at[cid, pl.ds(row0, ROWS_PER_SUB)])

    tail0 = NS * ROWS_PER_SUB
    tail = N - tail0

    @pl.when(sid == 0)
    def _():
        pltpu.sync_copy(acc_sh.at[pl.ds(tail0, tail)],
                        out_hbm.at[cid, pl.ds(tail0, tail)])


_sc_scatter = functools.partial(
    pl.kernel,
    mesh=plsc.VectorSubcoreMesh(core_axis_name="c", subcore_axis_name="s"),
    out_type=jax.ShapeDtypeStruct((NC, N, H), jnp.float32),
    scratch_types=[
        pltpu.VMEM((CH,), jnp.int32),
        pltpu.VMEM((CH,), jnp.int32),
        pltpu.VMEM((CH,), jnp.int32),
        pltpu.VMEM((CH,), jnp.int32),
        pltpu.VMEM((CH, 2 * H), jnp.float32),
        pltpu.VMEM((CH, 2 * H), jnp.float32),
        pltpu.VMEM((CH, H), jnp.float32),
        pltpu.VMEM((CH, H), jnp.float32),
        pltpu.VMEM_SHARED((N, H), jnp.float32),
        pltpu.SemaphoreType.DMA,
        pltpu.SemaphoreType.DMA,
        pltpu.SemaphoreType.DMA,
        pltpu.SemaphoreType.DMA,
        pltpu.SemaphoreType.DMA,
        pltpu.SemaphoreType.DMA,
    ],
)(_sc_body)


def kernel(node_feats, edge_feats, node_hidden, edge_index, W_node, W_edge,
           W_ih, W_hh, b_ih, b_hh):
    BN = 1000
    pr_arr = pl.pallas_call(
        _node_prep_body,
        grid=(N // BN,),
        in_specs=[pl.BlockSpec((BN, D), lambda i: (i, 0)),
                  pl.BlockSpec((BN, H), lambda i: (i, 0)),
                  pl.BlockSpec((H, D), lambda i: (0, 0))],
        out_specs=pl.BlockSpec((BN, 2 * H), lambda i: (i, 0)),
        out_shape=jax.ShapeDtypeStruct((N, 2 * H), jnp.float32),
    )(node_feats, node_hidden, W_node)

    BE = 4096
    ef_pad = jnp.pad(edge_feats, ((0, E_PAD - E), (0, 0)))
    q_arr = pl.pallas_call(
        _edge_prep_body,
        grid=(E_PAD // BE,),
        in_specs=[pl.BlockSpec((BE, DE), lambda i: (i, 0)),
                  pl.BlockSpec((H, DE), lambda i: (0, 0))],
        out_specs=pl.BlockSpec((BE, H), lambda i: (i, 0)),
        out_shape=jax.ShapeDtypeStruct((E_PAD, H), jnp.float32),
    )(ef_pad, W_edge)

    src_pad = jnp.concatenate(
        [edge_index[0], jnp.full((E_PAD - E,), N, jnp.int32)])
    dst_pad = jnp.concatenate(
        [edge_index[1], jnp.zeros((E_PAD - E,), jnp.int32)])
    ei_flat = jnp.concatenate([src_pad, dst_pad])
    pr_pad = jnp.concatenate(
        [pr_arr,
         jnp.concatenate([jnp.ones((N_PR - N, H), jnp.float32),
                          jnp.zeros((N_PR - N, H), jnp.float32)], axis=1)])

    zeros = jnp.zeros((N, H), jnp.float32)
    hp = _sc_scatter(pr_pad, q_arr, ei_flat, zeros)

    BG = 1000
    out = pl.pallas_call(
        _gru_body,
        grid=(N // BG,),
        in_specs=[pl.BlockSpec((2, BG, H), lambda i: (0, i, 0)),
                  pl.BlockSpec((BG, H), lambda i: (i, 0)),
                  pl.BlockSpec((3 * H, H), lambda i: (0, 0)),
                  pl.BlockSpec((3 * H, H), lambda i: (0, 0)),
                  pl.BlockSpec((1, 3 * H), lambda i: (0, 0)),
                  pl.BlockSpec((1, 3 * H), lambda i: (0, 0))],
        out_specs=pl.BlockSpec((BG, H), lambda i: (i, 0)),
        out_shape=jax.ShapeDtypeStruct((N, H), jnp.float32),
    )(hp, node_hidden, W_ih, W_hh, b_ih.reshape(1, 3 * H),
      b_hh.reshape(1, 3 * H))
    return out


# speedup vs baseline: 1.0295x; 1.0295x over previous
import functools
import jax
import jax.numpy as jnp
from jax import lax
from jax.experimental import pallas as pl
from jax.experimental.pallas import tpu as pltpu
from jax.experimental.pallas import tpu_sc as plsc

N, E, D, DE, H = 10000, 320000, 128, 16, 128
NC, NS, L = 2, 16, 16
NW = NC * NS
CH = 64
E_PAD = 323584
NCHUNKS = E_PAD // CH
CPW = NCHUNKS // NW
N_PR = N + 8
ROWS_PER_SUB = 624


def _node_prep_body(nf_ref, nh_ref, wn_ref, pr_ref):
    np_blk = lax.dot_general(nf_ref[...], wn_ref[...],
                             (((1,), (1,)), ((), ())),
                             preferred_element_type=jnp.float32)
    p = jnp.exp(np_blk - jnp.max(np_blk, axis=1, keepdims=True))
    pr_ref[:, :H] = p
    pr_ref[:, H:] = nh_ref[...] * p


def _edge_prep_body(ef_ref, we_ref, q_ref):
    ep = lax.dot_general(ef_ref[...], we_ref[...],
                         (((1,), (1,)), ((), ())),
                         preferred_element_type=jnp.float32)
    q_ref[...] = jnp.exp(ep - jnp.max(ep, axis=1, keepdims=True))


def _gru_body(hp_ref, nh_ref, wih_ref, whh_ref, bih_ref, bhh_ref, out_ref):
    h_new = hp_ref[0] + hp_ref[1]
    h = nh_ref[...]
    gi = lax.dot_general(h_new, wih_ref[...], (((1,), (1,)), ((), ())),
                         preferred_element_type=jnp.float32) + bih_ref[...]
    gh = lax.dot_general(h, whh_ref[...], (((1,), (1,)), ((), ())),
                         preferred_element_type=jnp.float32) + bhh_ref[...]
    r = jax.nn.sigmoid(gi[:, :H] + gh[:, :H])
    z = jax.nn.sigmoid(gi[:, H:2 * H] + gh[:, H:2 * H])
    n = jnp.tanh(gi[:, 2 * H:] + r * gh[:, 2 * H:])
    out_ref[...] = (1.0 - z) * n + z * h


def _sc_body(pr_hbm, q_hbm, ei_hbm, zero_hbm, out_hbm,
             src_a, dst_a, src_b, dst_b, pr_a, pr_b, q_a, q_b,
             acc_sh, sem_a, sem_b, sem_sa, sem_da, sem_sb, sem_db):
    cid = lax.axis_index("c")
    sid = lax.axis_index("s")
    wid = sid * NC + cid

    @pl.when(sid == 0)
    def _():
        pltpu.sync_copy(zero_hbm, acc_sh)

    plsc.subcore_barrier()

    def chunk_base(k):
        return (wid + k * NW) * CH

    def idx_src(k, ref, sem):
        pltpu.async_copy(ei_hbm.at[pl.ds(chunk_base(k), CH)], ref, sem)

    def idx_dst(k, ref, sem):
        pltpu.async_copy(ei_hbm.at[pl.ds(E_PAD + chunk_base(k), CH)], ref, sem)

    def wait_idx(ref, sem):
        pltpu.make_async_copy(ei_hbm.at[pl.ds(0, CH)], ref, sem).wait()

    def issue_gathers(k, src_v, pr_v, q_v, sem):
        pltpu.async_copy(pr_hbm.at[src_v], pr_v, sem)
        pltpu.async_copy(q_hbm.at[pl.ds(chunk_base(k), CH)], q_v, sem)

    def wait_gathers(src_v, pr_v, q_v, sem):
        pltpu.make_async_copy(pr_hbm.at[src_v], pr_v, sem).wait()
        pltpu.make_async_copy(q_hbm.at[pl.ds(0, CH)], q_v, sem).wait()

    lanes = lax.iota(jnp.int32, L)
    dnums = lax.GatherDimensionNumbers(
        offset_dims=(), collapsed_slice_dims=(0,), start_index_map=(0,))

    def compute_scatter(dst_v, pr_v, q_v):
        @plsc.parallel_loop(0, CH, unroll=1)
        def edge_body(i):
            qs = [q_v[i, pl.ds(j * L, L)] for j in range(H // L)]
            rqs = [pr_v[i, pl.ds(H + j * L, L)] * qs[j] for j in range(H // L)]
            acc = pr_v[i, pl.ds(0, L)] * qs[0]
            for j in range(1, H // L):
                acc = acc + pr_v[i, pl.ds(j * L, L)] * qs[j]
            for step in (1, 2, 4, 8):
                perm = lax.gather(
                    acc, (lanes ^ step).reshape(L, 1), dnums,
                    slice_sizes=(1,),
                    mode=lax.GatherScatterMode.PROMISE_IN_BOUNDS)
                acc = acc + perm
            s = 1.0 / acc
            for j in range(H // L):
                q_v[i, pl.ds(j * L, L)] = rqs[j] * s

        pltpu.sync_copy(q_v, acc_sh.at[dst_v], add=True)

    idx_src(0, src_a, sem_sa)
    idx_dst(0, dst_a, sem_da)
    wait_idx(src_a, sem_sa)
    issue_gathers(0, src_a, pr_a, q_a, sem_a)
    idx_src(1, src_b, sem_sb)
    idx_dst(1, dst_b, sem_db)

    def pair_body(kk, carry):
        k0 = 2 * kk
        wait_idx(src_b, sem_sb)
        issue_gathers(k0 + 1, src_b, pr_b, q_b, sem_b)
        wait_gathers(src_a, pr_a, q_a, sem_a)

        @pl.when(k0 + 2 < CPW)
        def _():
            idx_src(k0 + 2, src_a, sem_sa)

        wait_idx(dst_a, sem_da)
        compute_scatter(dst_a, pr_a, q_a)

        @pl.when(k0 + 2 < CPW)
        def _():
            idx_dst(k0 + 2, dst_a, sem_da)
            wait_idx(src_a, sem_sa)
            issue_gathers(k0 + 2, src_a, pr_a, q_a, sem_a)

        wait_gathers(src_b, pr_b, q_b, sem_b)

        @pl.when(k0 + 3 < CPW)
        def _():
            idx_src(k0 + 3, src_b, sem_sb)

        wait_idx(dst_b, sem_db)
        compute_scatter(dst_b, pr_b, q_b)

        @pl.when(k0 + 3 < CPW)
        def _():
            idx_dst(k0 + 3, dst_b, sem_db)

        return carry

    lax.fori_loop(0, CPW // 2, pair_body, 0)

    plsc.subcore_barrier()
    row0 = sid * ROWS_PER_SUB
    pltpu.sync_copy(acc_sh.at[pl.ds(row0, ROWS_PER_SUB)],
                    out_hbm.at[cid, pl.ds(row0, ROWS_PER_SUB)])

    tail0 = NS * ROWS_PER_SUB
    tail = N - tail0

    @pl.when(sid == 0)
    def _():
        pltpu.sync_copy(acc_sh.at[pl.ds(tail0, tail)],
                        out_hbm.at[cid, pl.ds(tail0, tail)])


_sc_scatter = functools.partial(
    pl.kernel,
    mesh=plsc.VectorSubcoreMesh(core_axis_name="c", subcore_axis_name="s"),
    out_type=jax.ShapeDtypeStruct((NC, N, H), jnp.float32),
    scratch_types=[
        pltpu.VMEM((CH,), jnp.int32),
        pltpu.VMEM((CH,), jnp.int32),
        pltpu.VMEM((CH,), jnp.int32),
        pltpu.VMEM((CH,), jnp.int32),
        pltpu.VMEM((CH, 2 * H), jnp.float32),
        pltpu.VMEM((CH, 2 * H), jnp.float32),
        pltpu.VMEM((CH, H), jnp.float32),
        pltpu.VMEM((CH, H), jnp.float32),
        pltpu.VMEM_SHARED((N, H), jnp.float32),
        pltpu.SemaphoreType.DMA,
        pltpu.SemaphoreType.DMA,
        pltpu.SemaphoreType.DMA,
        pltpu.SemaphoreType.DMA,
        pltpu.SemaphoreType.DMA,
        pltpu.SemaphoreType.DMA,
    ],
)(_sc_body)


def kernel(node_feats, edge_feats, node_hidden, edge_index, W_node, W_edge,
           W_ih, W_hh, b_ih, b_hh):
    BN = 1000
    pr_arr = pl.pallas_call(
        _node_prep_body,
        grid=(N // BN,),
        in_specs=[pl.BlockSpec((BN, D), lambda i: (i, 0)),
                  pl.BlockSpec((BN, H), lambda i: (i, 0)),
                  pl.BlockSpec((H, D), lambda i: (0, 0))],
        out_specs=pl.BlockSpec((BN, 2 * H), lambda i: (i, 0)),
        out_shape=jax.ShapeDtypeStruct((N, 2 * H), jnp.float32),
    )(node_feats, node_hidden, W_node)

    BE = 4096
    ef_pad = jnp.pad(edge_feats, ((0, E_PAD - E), (0, 0)))
    q_arr = pl.pallas_call(
        _edge_prep_body,
        grid=(E_PAD // BE,),
        in_specs=[pl.BlockSpec((BE, DE), lambda i: (i, 0)),
                  pl.BlockSpec((H, DE), lambda i: (0, 0))],
        out_specs=pl.BlockSpec((BE, H), lambda i: (i, 0)),
        out_shape=jax.ShapeDtypeStruct((E_PAD, H), jnp.float32),
    )(ef_pad, W_edge)

    src_pad = jnp.concatenate(
        [edge_index[0], jnp.full((E_PAD - E,), N, jnp.int32)])
    dst_pad = jnp.concatenate(
        [edge_index[1], jnp.zeros((E_PAD - E,), jnp.int32)])
    ei_flat = jnp.concatenate([src_pad, dst_pad])
    pr_pad = jnp.concatenate(
        [pr_arr,
         jnp.concatenate([jnp.ones((N_PR - N, H), jnp.float32),
                          jnp.zeros((N_PR - N, H), jnp.float32)], axis=1)])

    zeros = jnp.zeros((N, H), jnp.float32)
    hp = _sc_scatter(pr_pad, q_arr, ei_flat, zeros)

    BG = 1000
    out = pl.pallas_call(
        _gru_body,
        grid=(N // BG,),
        in_specs=[pl.BlockSpec((2, BG, H), lambda i: (0, i, 0)),
                  pl.BlockSpec((BG, H), lambda i: (i, 0)),
                  pl.BlockSpec((3 * H, H), lambda i: (0, 0)),
                  pl.BlockSpec((3 * H, H), lambda i: (0, 0)),
                  pl.BlockSpec((1, 3 * H), lambda i: (0, 0)),
                  pl.BlockSpec((1, 3 * H), lambda i: (0, 0))],
        out_specs=pl.BlockSpec((BG, H), lambda i: (i, 0)),
        out_shape=jax.ShapeDtypeStruct((N, H), jnp.float32),
    )(hp, node_hidden, W_ih, W_hh, b_ih.reshape(1, 3 * H),
      b_hh.reshape(1, 3 * H))
    return out


# speedup vs baseline: 1.2930x; 1.2560x over previous
import functools
import jax
import jax.numpy as jnp
from jax import lax
from jax.experimental import pallas as pl
from jax.experimental.pallas import tpu as pltpu
from jax.experimental.pallas import tpu_sc as plsc

N, E, D, DE, H = 10000, 320000, 128, 16, 128
NC, NS, L = 2, 16, 16
NW = NC * NS
CH = 64
NCHUNKS = E // CH
BASE_CHUNKS = NCHUNKS // NW
EXTRA = NCHUNKS % NW
ROWS_PER_SUB = 624


def _node_prep_body(nf_ref, nh_ref, wn_ref, pr_ref):
    np_blk = lax.dot_general(nf_ref[...], wn_ref[...],
                             (((1,), (1,)), ((), ())),
                             preferred_element_type=jnp.float32)
    p = jnp.exp(np_blk - jnp.max(np_blk, axis=1, keepdims=True))
    pr_ref[:, :H] = p
    pr_ref[:, H:] = nh_ref[...] * p


def _edge_prep_body(ef_ref, we_ref, q_ref):
    ep = lax.dot_general(ef_ref[...], we_ref[...],
                         (((1,), (1,)), ((), ())),
                         preferred_element_type=jnp.float32)
    q_ref[...] = jnp.exp(ep - jnp.max(ep, axis=1, keepdims=True))


def _gru_body(hp_ref, nh_ref, wih_ref, whh_ref, bih_ref, bhh_ref, out_ref):
    h_new = hp_ref[0] + hp_ref[1]
    h = nh_ref[...]
    gi = lax.dot_general(h_new, wih_ref[...], (((1,), (1,)), ((), ())),
                         preferred_element_type=jnp.float32) + bih_ref[...]
    gh = lax.dot_general(h, whh_ref[...], (((1,), (1,)), ((), ())),
                         preferred_element_type=jnp.float32) + bhh_ref[...]
    r = jax.nn.sigmoid(gi[:, :H] + gh[:, :H])
    z = jax.nn.sigmoid(gi[:, H:2 * H] + gh[:, H:2 * H])
    n = jnp.tanh(gi[:, 2 * H:] + r * gh[:, 2 * H:])
    out_ref[...] = (1.0 - z) * n + z * h


def _sc_body(pr_hbm, q_hbm, ei_hbm, zero_hbm, out_hbm,
             src_a, dst_a, src_b, dst_b, pr_a, pr_b, q_a, q_b,
             acc_sh, sem_a, sem_b):
    cid = lax.axis_index("c")
    sid = lax.axis_index("s")
    wid = sid * NC + cid

    @pl.when(sid == 0)
    def _():
        pltpu.sync_copy(zero_hbm, acc_sh)

    plsc.subcore_barrier()

    n_chunks = BASE_CHUNKS + jnp.where(wid < EXTRA, 1, 0)

    def chunk_base(k):
        return (wid + k * NW) * CH

    def load_idx(k, src_v, dst_v, sem):
        base = chunk_base(k)
        cp_s = pltpu.async_copy(ei_hbm.at[pl.ds(base, CH)], src_v, sem)
        cp_d = pltpu.async_copy(ei_hbm.at[pl.ds(E + base, CH)], dst_v, sem)
        cp_s.wait()
        cp_d.wait()

    def issue_gathers(k, src_v, pr_v, q_v, sem):
        pltpu.async_copy(pr_hbm.at[src_v], pr_v, sem)
        pltpu.async_copy(q_hbm.at[pl.ds(chunk_base(k), CH)], q_v, sem)

    def wait_gathers(src_v, pr_v, q_v, sem):
        pltpu.make_async_copy(pr_hbm.at[src_v], pr_v, sem).wait()
        pltpu.make_async_copy(q_hbm.at[pl.ds(0, CH)], q_v, sem).wait()

    lanes = lax.iota(jnp.int32, L)
    dnums = lax.GatherDimensionNumbers(
        offset_dims=(), collapsed_slice_dims=(0,), start_index_map=(0,))

    def compute_scatter(dst_v, pr_v, q_v):
        @plsc.parallel_loop(0, CH, unroll=1)
        def edge_body(i):
            qs = [q_v[i, pl.ds(j * L, L)] for j in range(H // L)]
            rqs = [pr_v[i, pl.ds(H + j * L, L)] * qs[j] for j in range(H // L)]
            acc = pr_v[i, pl.ds(0, L)] * qs[0]
            for j in range(1, H // L):
                acc = acc + pr_v[i, pl.ds(j * L, L)] * qs[j]
            for step in (1, 2, 4, 8):
                perm = lax.gather(
                    acc, (lanes ^ step).reshape(L, 1), dnums,
                    slice_sizes=(1,),
                    mode=lax.GatherScatterMode.PROMISE_IN_BOUNDS)
                acc = acc + perm
            s = 1.0 / acc
            for j in range(H // L):
                q_v[i, pl.ds(j * L, L)] = rqs[j] * s

        pltpu.sync_copy(q_v, acc_sh.at[dst_v], add=True)

    load_idx(0, src_a, dst_a, sem_a)
    issue_gathers(0, src_a, pr_a, q_a, sem_a)

    def pair_body(kk, carry):
        k0 = 2 * kk
        load_idx(k0 + 1, src_b, dst_b, sem_b)
        issue_gathers(k0 + 1, src_b, pr_b, q_b, sem_b)
        wait_gathers(src_a, pr_a, q_a, sem_a)
        compute_scatter(dst_a, pr_a, q_a)

        @pl.when(k0 + 2 < n_chunks)
        def _():
            load_idx(k0 + 2, src_a, dst_a, sem_a)
            issue_gathers(k0 + 2, src_a, pr_a, q_a, sem_a)

        wait_gathers(src_b, pr_b, q_b, sem_b)
        compute_scatter(dst_b, pr_b, q_b)
        return carry

    lax.fori_loop(0, BASE_CHUNKS // 2, pair_body, 0)

    @pl.when(n_chunks > BASE_CHUNKS)
    def _():
        wait_gathers(src_a, pr_a, q_a, sem_a)
        compute_scatter(dst_a, pr_a, q_a)

    plsc.subcore_barrier()
    row0 = sid * ROWS_PER_SUB
    pltpu.sync_copy(acc_sh.at[pl.ds(row0, ROWS_PER_SUB)],
                    out_hbm.at[cid, pl.ds(row0, ROWS_PER_SUB)])

    tail0 = NS * ROWS_PER_SUB
    tail = N - tail0

    @pl.when(sid == 0)
    def _():
        pltpu.sync_copy(acc_sh.at[pl.ds(tail0, tail)],
                        out_hbm.at[cid, pl.ds(tail0, tail)])


_sc_scatter = functools.partial(
    pl.kernel,
    mesh=plsc.VectorSubcoreMesh(core_axis_name="c", subcore_axis_name="s"),
    out_type=jax.ShapeDtypeStruct((NC, N, H), jnp.float32),
    scratch_types=[
        pltpu.VMEM((CH,), jnp.int32),
        pltpu.VMEM((CH,), jnp.int32),
        pltpu.VMEM((CH,), jnp.int32),
        pltpu.VMEM((CH,), jnp.int32),
        pltpu.VMEM((CH, 2 * H), jnp.float32),
        pltpu.VMEM((CH, 2 * H), jnp.float32),
        pltpu.VMEM((CH, H), jnp.float32),
        pltpu.VMEM((CH, H), jnp.float32),
        pltpu.VMEM_SHARED((N, H), jnp.float32),
        pltpu.SemaphoreType.DMA,
        pltpu.SemaphoreType.DMA,
    ],
)(_sc_body)


def kernel(node_feats, edge_feats, node_hidden, edge_index, W_node, W_edge,
           W_ih, W_hh, b_ih, b_hh):
    BN = 1000
    pr_arr = pl.pallas_call(
        _node_prep_body,
        grid=(N // BN,),
        in_specs=[pl.BlockSpec((BN, D), lambda i: (i, 0)),
                  pl.BlockSpec((BN, H), lambda i: (i, 0)),
                  pl.BlockSpec((H, D), lambda i: (0, 0))],
        out_specs=pl.BlockSpec((BN, 2 * H), lambda i: (i, 0)),
        out_shape=jax.ShapeDtypeStruct((N, 2 * H), jnp.float32),
    )(node_feats, node_hidden, W_node)

    BE = 8000
    q_arr = pl.pallas_call(
        _edge_prep_body,
        grid=(E // BE,),
        in_specs=[pl.BlockSpec((BE, DE), lambda i: (i, 0)),
                  pl.BlockSpec((H, DE), lambda i: (0, 0))],
        out_specs=pl.BlockSpec((BE, H), lambda i: (i, 0)),
        out_shape=jax.ShapeDtypeStruct((E, H), jnp.float32),
    )(edge_feats, W_edge)

    zeros = jnp.zeros((N, H), jnp.float32)
    hp = _sc_scatter(pr_arr, q_arr, edge_index.reshape(2 * E), zeros)

    BG = 1000
    out = pl.pallas_call(
        _gru_body,
        grid=(N // BG,),
        in_specs=[pl.BlockSpec((2, BG, H), lambda i: (0, i, 0)),
                  pl.BlockSpec((BG, H), lambda i: (i, 0)),
                  pl.BlockSpec((3 * H, H), lambda i: (0, 0)),
                  pl.BlockSpec((3 * H, H), lambda i: (0, 0)),
                  pl.BlockSpec((1, 3 * H), lambda i: (0, 0)),
                  pl.BlockSpec((1, 3 * H), lambda i: (0, 0))],
        out_specs=pl.BlockSpec((BG, H), lambda i: (i, 0)),
        out_shape=jax.ShapeDtypeStruct((N, H), jnp.float32),
    )(hp, node_hidden, W_ih, W_hh, b_ih.reshape(1, 3 * H),
      b_hh.reshape(1, 3 * H))
    return out


# speedup vs baseline: 1.2975x; 1.0035x over previous
import functools
import jax
import jax.numpy as jnp
from jax import lax
from jax.experimental import pallas as pl
from jax.experimental.pallas import tpu as pltpu
from jax.experimental.pallas import tpu_sc as plsc

N, E, D, DE, H = 10000, 320000, 128, 16, 128
NC, NS, L = 2, 16, 16
NW = NC * NS
CH = 64
NCHUNKS = E // CH
BASE_CHUNKS = NCHUNKS // NW
EXTRA = NCHUNKS % NW
ROWS_PER_SUB = 624


def _node_prep_body(nf_ref, nh_ref, wn_ref, pr_ref):
    np_blk = lax.dot_general(nf_ref[...], wn_ref[...],
                             (((1,), (1,)), ((), ())),
                             preferred_element_type=jnp.float32)
    p = jnp.exp(np_blk - jnp.max(np_blk, axis=1, keepdims=True))
    pr_ref[:, :H] = p
    pr_ref[:, H:] = nh_ref[...] * p


def _edge_prep_body(ef_ref, we_ref, q_ref):
    ep = lax.dot_general(ef_ref[...], we_ref[...],
                         (((1,), (1,)), ((), ())),
                         preferred_element_type=jnp.float32)
    q_ref[...] = jnp.exp(ep - jnp.max(ep, axis=1, keepdims=True))


def _gru_body(hp_ref, nh_ref, wih_ref, whh_ref, bih_ref, bhh_ref, out_ref):
    h_new = hp_ref[0] + hp_ref[1]
    h = nh_ref[...]
    gi = lax.dot_general(h_new, wih_ref[...], (((1,), (1,)), ((), ())),
                         preferred_element_type=jnp.float32) + bih_ref[...]
    gh = lax.dot_general(h, whh_ref[...], (((1,), (1,)), ((), ())),
                         preferred_element_type=jnp.float32) + bhh_ref[...]
    r = jax.nn.sigmoid(gi[:, :H] + gh[:, :H])
    z = jax.nn.sigmoid(gi[:, H:2 * H] + gh[:, H:2 * H])
    n = jnp.tanh(gi[:, 2 * H:] + r * gh[:, 2 * H:])
    out_ref[...] = (1.0 - z) * n + z * h


def _sc_body(pr_hbm, q_hbm, ei_hbm, zero_hbm, out_hbm,
             src_a, dst_a, src_b, dst_b, pr_a, pr_b, q_a, q_b,
             acc_sh, sem_a, sem_b):
    cid = lax.axis_index("c")
    sid = lax.axis_index("s")
    wid = sid * NC + cid

    zrow = sid * ROWS_PER_SUB
    pltpu.sync_copy(zero_hbm.at[pl.ds(zrow, ROWS_PER_SUB)],
                    acc_sh.at[pl.ds(zrow, ROWS_PER_SUB)])

    @pl.when(sid == 0)
    def _():
        ztail = NS * ROWS_PER_SUB
        pltpu.sync_copy(zero_hbm.at[pl.ds(ztail, N - ztail)],
                        acc_sh.at[pl.ds(ztail, N - ztail)])

    plsc.subcore_barrier()

    n_chunks = BASE_CHUNKS + jnp.where(wid < EXTRA, 1, 0)

    def chunk_base(k):
        return (wid + k * NW) * CH

    def load_idx(k, src_v, dst_v, sem):
        base = chunk_base(k)
        cp_s = pltpu.async_copy(ei_hbm.at[pl.ds(base, CH)], src_v, sem)
        cp_d = pltpu.async_copy(ei_hbm.at[pl.ds(E + base, CH)], dst_v, sem)
        cp_s.wait()
        cp_d.wait()

    def issue_gathers(k, src_v, pr_v, q_v, sem):
        pltpu.async_copy(pr_hbm.at[src_v], pr_v, sem)
        pltpu.async_copy(q_hbm.at[pl.ds(chunk_base(k), CH)], q_v, sem)

    def wait_gathers(src_v, pr_v, q_v, sem):
        pltpu.make_async_copy(pr_hbm.at[src_v], pr_v, sem).wait()
        pltpu.make_async_copy(q_hbm.at[pl.ds(0, CH)], q_v, sem).wait()

    lanes = lax.iota(jnp.int32, L)
    dnums = lax.GatherDimensionNumbers(
        offset_dims=(), collapsed_slice_dims=(0,), start_index_map=(0,))

    def compute_scatter(dst_v, pr_v, q_v):
        @plsc.parallel_loop(0, CH, unroll=1)
        def edge_body(i):
            qs = [q_v[i, pl.ds(j * L, L)] for j in range(H // L)]
            rqs = [pr_v[i, pl.ds(H + j * L, L)] * qs[j] for j in range(H // L)]
            acc = pr_v[i, pl.ds(0, L)] * qs[0]
            for j in range(1, H // L):
                acc = acc + pr_v[i, pl.ds(j * L, L)] * qs[j]
            for step in (1, 2, 4, 8):
                perm = lax.gather(
                    acc, (lanes ^ step).reshape(L, 1), dnums,
                    slice_sizes=(1,),
                    mode=lax.GatherScatterMode.PROMISE_IN_BOUNDS)
                acc = acc + perm
            s = 1.0 / acc
            for j in range(H // L):
                q_v[i, pl.ds(j * L, L)] = rqs[j] * s

        pltpu.sync_copy(q_v, acc_sh.at[dst_v], add=True)

    load_idx(0, src_a, dst_a, sem_a)
    issue_gathers(0, src_a, pr_a, q_a, sem_a)

    def pair_body(kk, carry):
        k0 = 2 * kk
        load_idx(k0 + 1, src_b, dst_b, sem_b)
        issue_gathers(k0 + 1, src_b, pr_b, q_b, sem_b)
        wait_gathers(src_a, pr_a, q_a, sem_a)
        compute_scatter(dst_a, pr_a, q_a)

        @pl.when(k0 + 2 < n_chunks)
        def _():
            load_idx(k0 + 2, src_a, dst_a, sem_a)
            issue_gathers(k0 + 2, src_a, pr_a, q_a, sem_a)

        wait_gathers(src_b, pr_b, q_b, sem_b)
        compute_scatter(dst_b, pr_b, q_b)
        return carry

    lax.fori_loop(0, BASE_CHUNKS // 2, pair_body, 0)

    @pl.when(n_chunks > BASE_CHUNKS)
    def _():
        wait_gathers(src_a, pr_a, q_a, sem_a)
        compute_scatter(dst_a, pr_a, q_a)

    plsc.subcore_barrier()
    row0 = sid * ROWS_PER_SUB
    pltpu.sync_copy(acc_sh.at[pl.ds(row0, ROWS_PER_SUB)],
                    out_hbm.at[cid, pl.ds(row0, ROWS_PER_SUB)])

    tail0 = NS * ROWS_PER_SUB
    tail = N - tail0

    @pl.when(sid == 0)
    def _():
        pltpu.sync_copy(acc_sh.at[pl.ds(tail0, tail)],
                        out_hbm.at[cid, pl.ds(tail0, tail)])


_sc_scatter = functools.partial(
    pl.kernel,
    mesh=plsc.VectorSubcoreMesh(core_axis_name="c", subcore_axis_name="s"),
    out_type=jax.ShapeDtypeStruct((NC, N, H), jnp.float32),
    scratch_types=[
        pltpu.VMEM((CH,), jnp.int32),
        pltpu.VMEM((CH,), jnp.int32),
        pltpu.VMEM((CH,), jnp.int32),
        pltpu.VMEM((CH,), jnp.int32),
        pltpu.VMEM((CH, 2 * H), jnp.float32),
        pltpu.VMEM((CH, 2 * H), jnp.float32),
        pltpu.VMEM((CH, H), jnp.float32),
        pltpu.VMEM((CH, H), jnp.float32),
        pltpu.VMEM_SHARED((N, H), jnp.float32),
        pltpu.SemaphoreType.DMA,
        pltpu.SemaphoreType.DMA,
    ],
)(_sc_body)


def kernel(node_feats, edge_feats, node_hidden, edge_index, W_node, W_edge,
           W_ih, W_hh, b_ih, b_hh):
    BN = 1000
    pr_arr = pl.pallas_call(
        _node_prep_body,
        grid=(N // BN,),
        in_specs=[pl.BlockSpec((BN, D), lambda i: (i, 0)),
                  pl.BlockSpec((BN, H), lambda i: (i, 0)),
                  pl.BlockSpec((H, D), lambda i: (0, 0))],
        out_specs=pl.BlockSpec((BN, 2 * H), lambda i: (i, 0)),
        out_shape=jax.ShapeDtypeStruct((N, 2 * H), jnp.float32),
    )(node_feats, node_hidden, W_node)

    BE = 16000
    q_arr = pl.pallas_call(
        _edge_prep_body,
        grid=(E // BE,),
        in_specs=[pl.BlockSpec((BE, DE), lambda i: (i, 0)),
                  pl.BlockSpec((H, DE), lambda i: (0, 0))],
        out_specs=pl.BlockSpec((BE, H), lambda i: (i, 0)),
        out_shape=jax.ShapeDtypeStruct((E, H), jnp.float32),
    )(edge_feats, W_edge)

    zeros = jnp.zeros((N, H), jnp.float32)
    hp = _sc_scatter(pr_arr, q_arr, edge_index.reshape(2 * E), zeros)

    BG = 1000
    out = pl.pallas_call(
        _gru_body,
        grid=(N // BG,),
        in_specs=[pl.BlockSpec((2, BG, H), lambda i: (0, i, 0)),
                  pl.BlockSpec((BG, H), lambda i: (i, 0)),
                  pl.BlockSpec((3 * H, H), lambda i: (0, 0)),
                  pl.BlockSpec((3 * H, H), lambda i: (0, 0)),
                  pl.BlockSpec((1, 3 * H), lambda i: (0, 0)),
                  pl.BlockSpec((1, 3 * H), lambda i: (0, 0))],
        out_specs=pl.BlockSpec((BG, H), lambda i: (i, 0)),
        out_shape=jax.ShapeDtypeStruct((N, H), jnp.float32),
    )(hp, node_hidden, W_ih, W_hh, b_ih.reshape(1, 3 * H),
      b_hh.reshape(1, 3 * H))
    return out
